# baseline (speedup 1.0000x reference)
import functools
import jax
import jax.numpy as jnp
from jax import lax
from jax.experimental import pallas as pl
from jax.experimental.pallas import tpu as pltpu, tpu_sc as plsc

N = 10000
E = 160000
D = 256
DE = 16
H = 512

NC, NS = 2, 16
NW = NC * NS
EPW = E // NW
GCH = 40
SCH = 40
NPS = N // NS
AW = 32


def _mm1_body(x_ref, wr_ref, wc_ref, b_ref, pr_ref, pc_ref):
    xb = x_ref[...]
    pr_ref[...] = jnp.dot(xb, wr_ref[...], preferred_element_type=jnp.float32) + b_ref[...]
    pc_ref[...] = jnp.dot(xb, wc_ref[...], preferred_element_type=jnp.float32)


def _precompute_tables(x, w_r, w_c, b1):
    blk = 2000
    grid = (N // blk,)
    return pl.pallas_call(
        _mm1_body,
        grid=grid,
        in_specs=[
            pl.BlockSpec((blk, D), lambda i: (i, 0)),
            pl.BlockSpec((D, H), lambda i: (0, 0)),
            pl.BlockSpec((D, H), lambda i: (0, 0)),
            pl.BlockSpec((1, H), lambda i: (0, 0)),
        ],
        out_specs=[
            pl.BlockSpec((blk, H), lambda i: (i, 0)),
            pl.BlockSpec((blk, H), lambda i: (i, 0)),
        ],
        out_shape=[
            jax.ShapeDtypeStruct((N, H), jnp.float32),
            jax.ShapeDtypeStruct((N, H), jnp.float32),
        ],
    )(x, w_r, w_c, b1.reshape(1, H))


def _gather_body(pr_hbm, pc_hbm, row_hbm, col_hbm, out_hbm,
                 idx_r, idx_c, gr, gc, sem):
    wid = lax.axis_index("s") * NC + lax.axis_index("c")
    base = wid * EPW

    def chunk(i, _):
        off = base + i * GCH
        pltpu.sync_copy(row_hbm.at[pl.ds(off, GCH)], idx_r)
        pltpu.sync_copy(col_hbm.at[pl.ds(off, GCH)], idx_c)
        pltpu.async_copy(pr_hbm.at[idx_r], gr, sem).wait()
        pltpu.async_copy(pc_hbm.at[idx_c], gc, sem).wait()

        def add_row(e, _):
            for kk in range(0, H, 16):
                sl = pl.ds(kk, 16)
                gr[e, sl] = gr[e, sl] + gc[e, sl]
            return 0

        lax.fori_loop(0, GCH, add_row, 0)
        pltpu.sync_copy(gr, out_hbm.at[pl.ds(off, GCH)])
        return 0

    lax.fori_loop(0, EPW // GCH, chunk, 0)


def _gather_add(p_r, p_c, row, col):
    mesh = plsc.VectorSubcoreMesh(core_axis_name="c", subcore_axis_name="s")
    return pl.kernel(
        _gather_body,
        out_type=jax.ShapeDtypeStruct((E, H), jnp.float32),
        mesh=mesh,
        scratch_types=[
            pltpu.VMEM((GCH,), jnp.int32),
            pltpu.VMEM((GCH,), jnp.int32),
            pltpu.VMEM((GCH, H), jnp.float32),
            pltpu.VMEM((GCH, H), jnp.float32),
            pltpu.SemaphoreType.DMA,
        ],
    )(p_r, p_c, row, col)


def _edge_body(g_ref, ea_ref, wa_ref, w2_ref, b2_ref, emb_ref, aug_ref):
    a = jnp.dot(ea_ref[...], wa_ref[...], preferred_element_type=jnp.float32)
    h = jax.nn.relu(g_ref[...] + a)
    emb = jnp.dot(h, w2_ref[...], preferred_element_type=jnp.float32) + b2_ref[...]
    emb_ref[...] = emb
    aug_ref[...] = jnp.concatenate(
        [emb, jnp.ones(emb.shape, dtype=jnp.float32)], axis=1)


def _edge_mlp(g, edge_attr, w_a, w2, b2):
    blk = 2000
    return pl.pallas_call(
        _edge_body,
        grid=(E // blk,),
        in_specs=[
            pl.BlockSpec((blk, H), lambda i: (i, 0)),
            pl.BlockSpec((blk, DE), lambda i: (i, 0)),
            pl.BlockSpec((DE, H), lambda i: (0, 0)),
            pl.BlockSpec((H, DE), lambda i: (0, 0)),
            pl.BlockSpec((1, DE), lambda i: (0, 0)),
        ],
        out_specs=[
            pl.BlockSpec((blk, DE), lambda i: (i, 0)),
            pl.BlockSpec((blk, AW), lambda i: (i, 0)),
        ],
        out_shape=[
            jax.ShapeDtypeStruct((E, DE), jnp.float32),
            jax.ShapeDtypeStruct((E, AW), jnp.float32),
        ],
    )(g, edge_attr, w_a, w2, b2.reshape(1, DE))


ST = 20
NPASS = 2
NPN = N // NPASS
APP = NPN * ST


def _scatter_body(aug_hbm, sidx_hbm, out_hbm, aug_v, idx_v, acc, sem):
    wid = lax.axis_index("s") * NC + lax.axis_index("c")
    base = wid * EPW

    lanes = lax.iota(jnp.int32, 16)
    for p in range(NPASS):
        lo = p * APP

        def zero(i, _):
            acc[pl.ds(i * 16, 16)] = jnp.zeros((16,), jnp.float32)
            return 0

        lax.fori_loop(0, (APP + 16) // 16, zero, 0)

        def chunk(i, _):
            off = base + i * SCH
            pltpu.sync_copy(aug_hbm.at[pl.ds(off, SCH)], aug_v)
            pltpu.sync_copy(sidx_hbm.at[pl.ds(off, SCH)], idx_v)

            def edge(e, _):
                for half in (0, 16):
                    idx = idx_v[e, pl.ds(half, 16)] - lo
                    val = aug_v[e, pl.ds(half, 16)]
                    m = (idx >= 0) & (idx < APP)
                    idx = jnp.where(m, idx, APP + lanes)
                    cur = plsc.load_gather(acc, [idx])
                    plsc.store_scatter(acc, [idx], cur + val)
                return 0

            lax.fori_loop(0, SCH, edge, 0)
            return 0

        lax.fori_loop(0, EPW // SCH, chunk, 0)
        pltpu.sync_copy(acc.at[pl.ds(0, APP)],
                        out_hbm.at[pl.ds(wid * (N * ST) + lo, APP)])


def _segment_sums(aug, sidx):
    mesh = plsc.VectorSubcoreMesh(core_axis_name="c", subcore_axis_name="s")
    return pl.kernel(
        _scatter_body,
        out_type=jax.ShapeDtypeStruct((NW * N * ST,), jnp.float32),
        mesh=mesh,
        compiler_params=pltpu.CompilerParams(needs_layout_passes=False),
        scratch_types=[
            pltpu.VMEM((SCH, AW), jnp.float32),
            pltpu.VMEM((SCH, AW), jnp.int32),
            pltpu.VMEM((APP + 16,), jnp.float32),
            pltpu.SemaphoreType.DMA,
        ],
    )(aug, sidx)


def _node_body(x_ref, p_ref, w1x_ref, w1a_ref, b1_ref, w2_ref, b2_ref, o_ref):
    parts = jnp.sum(p_ref[...], axis=0)
    sums = parts[:, :DE]
    cnts = parts[:, DE:DE + 1]
    agg = sums / jnp.maximum(cnts, 1.0)
    h = jax.nn.relu(
        jnp.dot(x_ref[...], w1x_ref[...], preferred_element_type=jnp.float32)
        + jnp.dot(agg, w1a_ref[...], preferred_element_type=jnp.float32)
        + b1_ref[...])
    o_ref[...] = jnp.dot(h, w2_ref[...], preferred_element_type=jnp.float32) + b2_ref[...]


def _node_mlp(x, parts, w1x, w1a, b1, w2, b2):
    blk = 1000
    return pl.pallas_call(
        _node_body,
        grid=(N // blk,),
        in_specs=[
            pl.BlockSpec((blk, D), lambda i: (i, 0)),
            pl.BlockSpec((NW, blk, ST), lambda i: (0, i, 0)),
            pl.BlockSpec((D, H), lambda i: (0, 0)),
            pl.BlockSpec((DE, H), lambda i: (0, 0)),
            pl.BlockSpec((1, H), lambda i: (0, 0)),
            pl.BlockSpec((H, D), lambda i: (0, 0)),
            pl.BlockSpec((1, D), lambda i: (0, 0)),
        ],
        out_specs=pl.BlockSpec((blk, D), lambda i: (i, 0)),
        out_shape=jax.ShapeDtypeStruct((N, D), jnp.float32),
    )(x, parts, w1x, w1a, b1.reshape(1, H), w2, b2.reshape(1, D))


@jax.jit
def kernel(x, edge_index, edge_attr, W1_e, b1_e, W2_e, b2_e,
           W1_n, b1_n, W2_n, b2_n):
    x = x.astype(jnp.float32)
    edge_attr = edge_attr.astype(jnp.float32)
    row = edge_index[0]
    col = edge_index[1]
    w_r = W1_e[:D]
    w_c = W1_e[D:2 * D]
    w_a = W1_e[2 * D:]

    cols = jnp.concatenate([jnp.arange(DE + 1, dtype=jnp.int32),
                            jnp.full((AW - DE - 1,), 1 << 30, jnp.int32)])
    sidx = row[:, None].astype(jnp.int32) * ST + cols[None, :]

    p_r, p_c = _precompute_tables(x, w_r, w_c, b1_e)
    g = _gather_add(p_r, p_c, row, col)
    emb, aug = _edge_mlp(g, edge_attr, w_a, W2_e, b2_e)
    parts = _segment_sums(aug, sidx).reshape(NW, N, ST)
    node = _node_mlp(x, parts, W1_n[:D], W1_n[D:], b1_n, W2_n, b2_n)
    return emb, node


# speedup vs baseline: 1.1561x; 1.1561x over previous
import functools
import jax
import jax.numpy as jnp
from jax import lax
from jax.experimental import pallas as pl
from jax.experimental.pallas import tpu as pltpu, tpu_sc as plsc

N = 10000
E = 160000
D = 256
DE = 16
H = 512

NC, NS = 2, 16
NW = NC * NS
EPW = E // NW
GCH = 96
SCH = 40
NPS = N // NS
AW = 32


def _mm1_body(x_ref, wr_ref, wc_ref, b_ref, pr_ref, pc_ref):
    xb = x_ref[...]
    pr = jnp.dot(xb, wr_ref[...], preferred_element_type=jnp.float32) + b_ref[...]
    pc = jnp.dot(xb, wc_ref[...], preferred_element_type=jnp.float32)
    pr_ref[...] = pr
    pc_ref[...] = pc


def _precompute_tables(x, w_r, w_c, b1):
    blk = 2000
    grid = (N // blk,)
    return pl.pallas_call(
        _mm1_body,
        grid=grid,
        in_specs=[
            pl.BlockSpec((blk, D), lambda i: (i, 0)),
            pl.BlockSpec((D, H), lambda i: (0, 0)),
            pl.BlockSpec((D, H), lambda i: (0, 0)),
            pl.BlockSpec((1, H), lambda i: (0, 0)),
        ],
        out_specs=[
            pl.BlockSpec((blk, H), lambda i: (i, 0)),
            pl.BlockSpec((blk, H), lambda i: (i, 0)),
        ],
        out_shape=[
            jax.ShapeDtypeStruct((N, H), jnp.float32),
            jax.ShapeDtypeStruct((N, H), jnp.float32),
        ],
    )(x, w_r, w_c, b1.reshape(1, H))


GTAIL = EPW - (EPW // GCH) * GCH


def _gather_body(pr_hbm, pc_hbm, row_hbm, col_hbm, out_hbm,
                 idx_r, idx_c, gr, gc, sem):
    wid = lax.axis_index("s") * NC + lax.axis_index("c")
    base = wid * EPW
    pltpu.sync_copy(row_hbm.at[pl.ds(base, EPW)], idx_r)
    pltpu.sync_copy(col_hbm.at[pl.ds(base, EPW)], idx_c)

    def do_chunk(o, rows):
        dr = gr.at[pl.ds(0, rows)]
        dc = gc.at[pl.ds(0, rows)]
        pltpu.async_copy(pr_hbm.at[idx_r.at[pl.ds(o, rows)]], dr, sem)
        pltpu.async_copy(pc_hbm.at[idx_c.at[pl.ds(o, rows)]], dc, sem).wait()
        pltpu.make_async_copy(pr_hbm.at[idx_r.at[pl.ds(o, rows)]], dr,
                              sem).wait()

        def add_row(e, _):
            for kk in range(0, H, 16):
                sl = pl.ds(kk, 16)
                gr[e, sl] = gr[e, sl] + gc[e, sl]
            return 0

        lax.fori_loop(0, rows, add_row, 0)
        pltpu.sync_copy(dr, out_hbm.at[pl.ds(base + o, rows)])

    def chunk(i, _):
        do_chunk(i * GCH, GCH)
        return 0

    lax.fori_loop(0, EPW // GCH, chunk, 0)
    if GTAIL:
        do_chunk((EPW // GCH) * GCH, GTAIL)


def _gather_add(p_r, p_c, row, col):
    mesh = plsc.VectorSubcoreMesh(core_axis_name="c", subcore_axis_name="s")
    return pl.kernel(
        _gather_body,
        out_type=jax.ShapeDtypeStruct((E, H), jnp.float32),
        mesh=mesh,
        scratch_types=[
            pltpu.VMEM((EPW,), jnp.int32),
            pltpu.VMEM((EPW,), jnp.int32),
            pltpu.VMEM((GCH, H), jnp.float32),
            pltpu.VMEM((GCH, H), jnp.float32),
            pltpu.SemaphoreType.DMA,
        ],
    )(p_r, p_c, row, col)


def _edge_body(g_ref, ea_ref, wa_ref, w2_ref, b2_ref, emb_ref, aug_ref):
    a = jnp.dot(ea_ref[...], wa_ref[...], preferred_element_type=jnp.float32)
    h = jax.nn.relu(g_ref[...] + a)
    emb = jnp.dot(h, w2_ref[...], preferred_element_type=jnp.float32) + b2_ref[...]
    emb_ref[...] = emb
    aug_ref[...] = jnp.concatenate(
        [emb, jnp.ones(emb.shape, dtype=jnp.float32)], axis=1)


def _edge_mlp(g, edge_attr, w_a, w2, b2):
    blk = 2000
    return pl.pallas_call(
        _edge_body,
        grid=(E // blk,),
        in_specs=[
            pl.BlockSpec((blk, H), lambda i: (i, 0)),
            pl.BlockSpec((blk, DE), lambda i: (i, 0)),
            pl.BlockSpec((DE, H), lambda i: (0, 0)),
            pl.BlockSpec((H, DE), lambda i: (0, 0)),
            pl.BlockSpec((1, DE), lambda i: (0, 0)),
        ],
        out_specs=[
            pl.BlockSpec((blk, DE), lambda i: (i, 0)),
            pl.BlockSpec((blk, AW), lambda i: (i, 0)),
        ],
        out_shape=[
            jax.ShapeDtypeStruct((E, DE), jnp.float32),
            jax.ShapeDtypeStruct((E, AW), jnp.float32),
        ],
    )(g, edge_attr, w_a, w2, b2.reshape(1, DE))


ST = 20
NPASS = 2
NPN = N // NPASS
APP = NPN * ST


def _scatter_body(aug_hbm, sidx_hbm, out_hbm, aug_v, idx_v, acc, sem):
    wid = lax.axis_index("s") * NC + lax.axis_index("c")
    base = wid * EPW

    lanes = lax.iota(jnp.int32, 16)
    for p in range(NPASS):
        lo = p * APP

        def zero(i, _):
            acc[pl.ds(i * 16, 16)] = jnp.zeros((16,), jnp.float32)
            return 0

        lax.fori_loop(0, (APP + 16) // 16, zero, 0)

        def chunk(i, _):
            off = base + i * SCH
            pltpu.sync_copy(aug_hbm.at[pl.ds(off, SCH)], aug_v)
            pltpu.sync_copy(sidx_hbm.at[pl.ds(off, SCH)], idx_v)

            def edge(e4, _):
                for j in range(4):
                    e = e4 * 4 + j
                    for half in (0, 16):
                        rel = idx_v[e, pl.ds(half, 16)] - lo
                        val = aug_v[e, pl.ds(half, 16)]
                        m = plsc.bitcast(rel, jnp.uint32) < APP
                        idx = jnp.where(m, rel, APP + lanes)
                        cur = plsc.load_gather(acc, [idx])
                        plsc.store_scatter(acc, [idx], cur + val)
                return 0

            lax.fori_loop(0, SCH // 4, edge, 0)
            return 0

        lax.fori_loop(0, EPW // SCH, chunk, 0)
        pltpu.sync_copy(acc.at[pl.ds(0, APP)],
                        out_hbm.at[pl.ds(wid * (N * ST) + lo, APP)])


def _segment_sums(aug, sidx):
    mesh = plsc.VectorSubcoreMesh(core_axis_name="c", subcore_axis_name="s")
    return pl.kernel(
        _scatter_body,
        out_type=jax.ShapeDtypeStruct((NW * N * ST,), jnp.float32),
        mesh=mesh,
        compiler_params=pltpu.CompilerParams(needs_layout_passes=False),
        scratch_types=[
            pltpu.VMEM((SCH, AW), jnp.float32),
            pltpu.VMEM((SCH, AW), jnp.int32),
            pltpu.VMEM((APP + 16,), jnp.float32),
            pltpu.SemaphoreType.DMA,
        ],
    )(aug, sidx)


def _node_body(x_ref, p_ref, w1x_ref, w1a_ref, b1_ref, w2_ref, b2_ref, o_ref):
    parts = jnp.sum(p_ref[...], axis=0)
    sums = parts[:, :DE]
    cnts = parts[:, DE:DE + 1]
    agg = sums / jnp.maximum(cnts, 1.0)
    h = jax.nn.relu(
        jnp.dot(x_ref[...], w1x_ref[...], preferred_element_type=jnp.float32)
        + jnp.dot(agg, w1a_ref[...], preferred_element_type=jnp.float32)
        + b1_ref[...])
    o_ref[...] = jnp.dot(h, w2_ref[...], preferred_element_type=jnp.float32) + b2_ref[...]


def _node_mlp(x, parts, w1x, w1a, b1, w2, b2):
    blk = 1000
    return pl.pallas_call(
        _node_body,
        grid=(N // blk,),
        in_specs=[
            pl.BlockSpec((blk, D), lambda i: (i, 0)),
            pl.BlockSpec((NW, blk, ST), lambda i: (0, i, 0)),
            pl.BlockSpec((D, H), lambda i: (0, 0)),
            pl.BlockSpec((DE, H), lambda i: (0, 0)),
            pl.BlockSpec((1, H), lambda i: (0, 0)),
            pl.BlockSpec((H, D), lambda i: (0, 0)),
            pl.BlockSpec((1, D), lambda i: (0, 0)),
        ],
        out_specs=pl.BlockSpec((blk, D), lambda i: (i, 0)),
        out_shape=jax.ShapeDtypeStruct((N, D), jnp.float32),
    )(x, parts, w1x, w1a, b1.reshape(1, H), w2, b2.reshape(1, D))


@jax.jit
def kernel(x, edge_index, edge_attr, W1_e, b1_e, W2_e, b2_e,
           W1_n, b1_n, W2_n, b2_n):
    x = x.astype(jnp.float32)
    edge_attr = edge_attr.astype(jnp.float32)
    row = edge_index[0]
    col = edge_index[1]
    w_r = W1_e[:D]
    w_c = W1_e[D:2 * D]
    w_a = W1_e[2 * D:]

    cols = jnp.concatenate([jnp.arange(DE + 1, dtype=jnp.int32),
                            jnp.full((AW - DE - 1,), 1 << 30, jnp.int32)])
    sidx = row[:, None].astype(jnp.int32) * ST + cols[None, :]

    p_r, p_c = _precompute_tables(x, w_r, w_c, b1_e)
    g = _gather_add(p_r, p_c, row, col)
    emb, aug = _edge_mlp(g, edge_attr, w_a, W2_e, b2_e)
    parts = _segment_sums(aug, sidx).reshape(NW, N, ST)
    node = _node_mlp(x, parts, W1_n[:D], W1_n[D:], b1_n, W2_n, b2_n)
    return emb, node


# speedup vs baseline: 1.2000x; 1.0380x over previous
import functools
import jax
import jax.numpy as jnp
from jax import lax
from jax.experimental import pallas as pl
from jax.experimental.pallas import tpu as pltpu, tpu_sc as plsc

N = 10000
E = 160000
D = 256
DE = 16
H = 512

NC, NS = 2, 16
NW = NC * NS
EPW = E // NW
GCH = 96
AW = 32


def _mm1_body(x_ref, wr_ref, wc_ref, b_ref, pr_ref, pc_ref):
    xb = x_ref[...]
    pr = jnp.dot(xb, wr_ref[...], preferred_element_type=jnp.float32) + b_ref[...]
    pc = jnp.dot(xb, wc_ref[...], preferred_element_type=jnp.float32)
    pr_ref[...] = pr
    pc_ref[...] = pc


def _precompute_tables(x, w_r, w_c, b1):
    blk = 2000
    grid = (N // blk,)
    return pl.pallas_call(
        _mm1_body,
        grid=grid,
        in_specs=[
            pl.BlockSpec((blk, D), lambda i: (i, 0)),
            pl.BlockSpec((D, H), lambda i: (0, 0)),
            pl.BlockSpec((D, H), lambda i: (0, 0)),
            pl.BlockSpec((1, H), lambda i: (0, 0)),
        ],
        out_specs=[
            pl.BlockSpec((blk, H), lambda i: (i, 0)),
            pl.BlockSpec((blk, H), lambda i: (i, 0)),
        ],
        out_shape=[
            jax.ShapeDtypeStruct((N, H), jnp.float32),
            jax.ShapeDtypeStruct((N, H), jnp.float32),
        ],
    )(x, w_r, w_c, b1.reshape(1, H))


GTAIL = EPW - (EPW // GCH) * GCH


def _gather_body(pr_hbm, pc_hbm, row_hbm, col_hbm, out_hbm,
                 idx_r, idx_c, gr, gc, sem):
    wid = lax.axis_index("s") * NC + lax.axis_index("c")
    base = wid * EPW
    pltpu.sync_copy(row_hbm.at[pl.ds(base, EPW)], idx_r)
    pltpu.sync_copy(col_hbm.at[pl.ds(base, EPW)], idx_c)

    def do_chunk(o, rows):
        dr = gr.at[pl.ds(0, rows)]
        dc = gc.at[pl.ds(0, rows)]
        pltpu.async_copy(pr_hbm.at[idx_r.at[pl.ds(o, rows)]], dr, sem)
        pltpu.async_copy(pc_hbm.at[idx_c.at[pl.ds(o, rows)]], dc, sem).wait()
        pltpu.make_async_copy(pr_hbm.at[idx_r.at[pl.ds(o, rows)]], dr,
                              sem).wait()

        def add_row(e, _):
            for kk in range(0, H, 16):
                sl = pl.ds(kk, 16)
                gr[e, sl] = gr[e, sl] + gc[e, sl]
            return 0

        lax.fori_loop(0, rows, add_row, 0)
        pltpu.sync_copy(dr, out_hbm.at[pl.ds(base + o, rows)])

    def chunk(i, _):
        do_chunk(i * GCH, GCH)
        return 0

    lax.fori_loop(0, EPW // GCH, chunk, 0)
    if GTAIL:
        do_chunk((EPW // GCH) * GCH, GTAIL)


def _gather_add(p_r, p_c, row, col):
    mesh = plsc.VectorSubcoreMesh(core_axis_name="c", subcore_axis_name="s")
    return pl.kernel(
        _gather_body,
        out_type=jax.ShapeDtypeStruct((E, H), jnp.float32),
        mesh=mesh,
        scratch_types=[
            pltpu.VMEM((EPW,), jnp.int32),
            pltpu.VMEM((EPW,), jnp.int32),
            pltpu.VMEM((GCH, H), jnp.float32),
            pltpu.VMEM((GCH, H), jnp.float32),
            pltpu.SemaphoreType.DMA,
        ],
    )(p_r, p_c, row, col)


def _edge_body(g_ref, ea_ref, wa_ref, w2_ref, b2_ref, emb_ref, aug_ref):
    a = jnp.dot(ea_ref[...], wa_ref[...], preferred_element_type=jnp.float32)
    h = jax.nn.relu(g_ref[...] + a)
    emb = jnp.dot(h, w2_ref[...], preferred_element_type=jnp.float32) + b2_ref[...]
    emb_ref[...] = emb
    aug_ref[...] = jnp.concatenate(
        [emb, jnp.ones(emb.shape, dtype=jnp.float32)], axis=1)


def _edge_mlp(g, edge_attr, w_a, w2, b2):
    blk = 2000
    return pl.pallas_call(
        _edge_body,
        grid=(E // blk,),
        in_specs=[
            pl.BlockSpec((blk, H), lambda i: (i, 0)),
            pl.BlockSpec((blk, DE), lambda i: (i, 0)),
            pl.BlockSpec((DE, H), lambda i: (0, 0)),
            pl.BlockSpec((H, DE), lambda i: (0, 0)),
            pl.BlockSpec((1, DE), lambda i: (0, 0)),
        ],
        out_specs=[
            pl.BlockSpec((blk, DE), lambda i: (i, 0)),
            pl.BlockSpec((blk, AW), lambda i: (i, 0)),
        ],
        out_shape=[
            jax.ShapeDtypeStruct((E, DE), jnp.float32),
            jax.ShapeDtypeStruct((E, AW), jnp.float32),
        ],
    )(g, edge_attr, w_a, w2, b2.reshape(1, DE))


ST = 20
NPASS = 2
NPN = N // NPASS
APP = NPN * ST
SCH = 40


def _scatter_body(aug_hbm, sidx_hbm, out_hbm, aug_v, idx_v, acc, sem):
    wid = lax.axis_index("s") * NC + lax.axis_index("c")
    base = wid * EPW

    lanes = lax.iota(jnp.int32, 16)
    for p in range(NPASS):
        lo = p * APP

        def zero(i, _):
            acc[pl.ds(i * 16, 16)] = jnp.zeros((16,), jnp.float32)
            return 0

        lax.fori_loop(0, (APP + 16) // 16, zero, 0)

        def chunk(i, _):
            off = base + i * SCH
            pltpu.sync_copy(aug_hbm.at[pl.ds(off, SCH)], aug_v)
            pltpu.sync_copy(sidx_hbm.at[pl.ds(off, SCH)], idx_v)

            def edge(e4, _):
                for j in range(4):
                    e = e4 * 4 + j
                    for half in (0, 16):
                        rel = idx_v[e, pl.ds(half, 16)] - lo
                        val = aug_v[e, pl.ds(half, 16)]
                        m = plsc.bitcast(rel, jnp.uint32) < APP
                        idx = jnp.where(m, rel, APP + lanes)
                        plsc.addupdate_scatter(acc, [idx], val)
                return 0

            lax.fori_loop(0, SCH // 4, edge, 0)
            return 0

        lax.fori_loop(0, EPW // SCH, chunk, 0)
        pltpu.sync_copy(acc.at[pl.ds(0, APP)],
                        out_hbm.at[pl.ds(wid * (N * ST) + lo, APP)])


def _segment_sums(aug, sidx):
    mesh = plsc.VectorSubcoreMesh(core_axis_name="c", subcore_axis_name="s")
    return pl.kernel(
        _scatter_body,
        out_type=jax.ShapeDtypeStruct((NW * N * ST,), jnp.float32),
        mesh=mesh,
        compiler_params=pltpu.CompilerParams(needs_layout_passes=False),
        scratch_types=[
            pltpu.VMEM((SCH, AW), jnp.float32),
            pltpu.VMEM((SCH, AW), jnp.int32),
            pltpu.VMEM((APP + 16,), jnp.float32),
            pltpu.SemaphoreType.DMA,
        ],
    )(aug, sidx)


def _node_body(x_ref, p_ref, w1x_ref, w1a_ref, b1_ref, w2_ref, b2_ref, o_ref):
    parts = jnp.sum(p_ref[...], axis=0)
    sums = parts[:, :DE]
    cnts = parts[:, DE:DE + 1]
    agg = sums / jnp.maximum(cnts, 1.0)
    h = jax.nn.relu(
        jnp.dot(x_ref[...], w1x_ref[...], preferred_element_type=jnp.float32)
        + jnp.dot(agg, w1a_ref[...], preferred_element_type=jnp.float32)
        + b1_ref[...])
    o_ref[...] = jnp.dot(h, w2_ref[...], preferred_element_type=jnp.float32) + b2_ref[...]


def _node_mlp(x, parts, w1x, w1a, b1, w2, b2):
    blk = 1000
    return pl.pallas_call(
        _node_body,
        grid=(N // blk,),
        in_specs=[
            pl.BlockSpec((blk, D), lambda i: (i, 0)),
            pl.BlockSpec((NW, blk, ST), lambda i: (0, i, 0)),
            pl.BlockSpec((D, H), lambda i: (0, 0)),
            pl.BlockSpec((DE, H), lambda i: (0, 0)),
            pl.BlockSpec((1, H), lambda i: (0, 0)),
            pl.BlockSpec((H, D), lambda i: (0, 0)),
            pl.BlockSpec((1, D), lambda i: (0, 0)),
        ],
        out_specs=pl.BlockSpec((blk, D), lambda i: (i, 0)),
        out_shape=jax.ShapeDtypeStruct((N, D), jnp.float32),
    )(x, parts, w1x, w1a, b1.reshape(1, H), w2, b2.reshape(1, D))


@jax.jit
def kernel(x, edge_index, edge_attr, W1_e, b1_e, W2_e, b2_e,
           W1_n, b1_n, W2_n, b2_n):
    x = x.astype(jnp.float32)
    edge_attr = edge_attr.astype(jnp.float32)
    row = edge_index[0]
    col = edge_index[1]
    w_r = W1_e[:D]
    w_c = W1_e[D:2 * D]
    w_a = W1_e[2 * D:]

    cols = jnp.concatenate([jnp.arange(DE + 1, dtype=jnp.int32),
                            jnp.full((AW - DE - 1,), 1 << 30, jnp.int32)])
    sidx = row[:, None].astype(jnp.int32) * ST + cols[None, :]

    p_r, p_c = _precompute_tables(x, w_r, w_c, b1_e)
    g = _gather_add(p_r, p_c, row, col)
    emb, aug = _edge_mlp(g, edge_attr, w_a, W2_e, b2_e)
    parts = _segment_sums(aug, sidx).reshape(NW, N, ST)
    node = _node_mlp(x, parts, W1_n[:D], W1_n[D:], b1_n, W2_n, b2_n)
    return emb, node


# speedup vs baseline: 1.2081x; 1.0067x over previous
import functools
import jax
import jax.numpy as jnp
from jax import lax
from jax.experimental import pallas as pl
from jax.experimental.pallas import tpu as pltpu, tpu_sc as plsc

N = 10000
E = 160000
D = 256
DE = 16
H = 512

NC, NS = 2, 16
NW = NC * NS
EPW = E // NW
GCH = 96
AW = 32


def _mm1_body(x_ref, wr_ref, wc_ref, b_ref, pr_ref, pc_ref):
    xb = x_ref[...]
    pr = jnp.dot(xb, wr_ref[...], preferred_element_type=jnp.float32) + b_ref[...]
    pc = jnp.dot(xb, wc_ref[...], preferred_element_type=jnp.float32)
    pr_ref[...] = pr
    pc_ref[...] = pc


def _precompute_tables(x, w_r, w_c, b1):
    blk = 2000
    grid = (N // blk,)
    return pl.pallas_call(
        _mm1_body,
        grid=grid,
        in_specs=[
            pl.BlockSpec((blk, D), lambda i: (i, 0)),
            pl.BlockSpec((D, H), lambda i: (0, 0)),
            pl.BlockSpec((D, H), lambda i: (0, 0)),
            pl.BlockSpec((1, H), lambda i: (0, 0)),
        ],
        out_specs=[
            pl.BlockSpec((blk, H), lambda i: (i, 0)),
            pl.BlockSpec((blk, H), lambda i: (i, 0)),
        ],
        out_shape=[
            jax.ShapeDtypeStruct((N, H), jnp.float32),
            jax.ShapeDtypeStruct((N, H), jnp.float32),
        ],
    )(x, w_r, w_c, b1.reshape(1, H))


GTAIL = EPW - (EPW // GCH) * GCH


def _gather_body(pr_hbm, pc_hbm, row_hbm, col_hbm, out_hbm,
                 idx_r, idx_c, gr, gc, sem):
    wid = lax.axis_index("s") * NC + lax.axis_index("c")
    base = wid * EPW
    pltpu.sync_copy(row_hbm.at[pl.ds(base, EPW)], idx_r)
    pltpu.sync_copy(col_hbm.at[pl.ds(base, EPW)], idx_c)

    def do_chunk(o, rows):
        dr = gr.at[pl.ds(0, rows)]
        dc = gc.at[pl.ds(0, rows)]
        pltpu.async_copy(pr_hbm.at[idx_r.at[pl.ds(o, rows)]], dr, sem)
        pltpu.async_copy(pc_hbm.at[idx_c.at[pl.ds(o, rows)]], dc, sem).wait()
        pltpu.make_async_copy(pr_hbm.at[idx_r.at[pl.ds(o, rows)]], dr,
                              sem).wait()

        def add_row(e, _):
            for kk in range(0, H, 16):
                sl = pl.ds(kk, 16)
                gr[e, sl] = gr[e, sl] + gc[e, sl]
            return 0

        lax.fori_loop(0, rows, add_row, 0)
        pltpu.sync_copy(dr, out_hbm.at[pl.ds(base + o, rows)])

    def chunk(i, _):
        do_chunk(i * GCH, GCH)
        return 0

    lax.fori_loop(0, EPW // GCH, chunk, 0)
    if GTAIL:
        do_chunk((EPW // GCH) * GCH, GTAIL)


def _gather_add(p_r, p_c, row, col):
    mesh = plsc.VectorSubcoreMesh(core_axis_name="c", subcore_axis_name="s")
    return pl.kernel(
        _gather_body,
        out_type=jax.ShapeDtypeStruct((E, H), jnp.float32),
        mesh=mesh,
        scratch_types=[
            pltpu.VMEM((EPW,), jnp.int32),
            pltpu.VMEM((EPW,), jnp.int32),
            pltpu.VMEM((GCH, H), jnp.float32),
            pltpu.VMEM((GCH, H), jnp.float32),
            pltpu.SemaphoreType.DMA,
        ],
    )(p_r, p_c, row, col)


def _edge_body(g_ref, ea_ref, wa_ref, w2_ref, b2_ref, emb_ref, aug_ref):
    a = jnp.dot(ea_ref[...].astype(jnp.bfloat16), wa_ref[...].astype(jnp.bfloat16),
                preferred_element_type=jnp.float32)
    h = jax.nn.relu(g_ref[...] + a)
    emb = jnp.dot(h.astype(jnp.bfloat16), w2_ref[...].astype(jnp.bfloat16),
                  preferred_element_type=jnp.float32) + b2_ref[...]
    emb_ref[...] = emb
    aug_ref[...] = jnp.concatenate(
        [emb, jnp.ones(emb.shape, dtype=jnp.float32)], axis=1)


def _edge_mlp(g, edge_attr, w_a, w2, b2):
    blk = 2000
    return pl.pallas_call(
        _edge_body,
        grid=(E // blk,),
        in_specs=[
            pl.BlockSpec((blk, H), lambda i: (i, 0)),
            pl.BlockSpec((blk, DE), lambda i: (i, 0)),
            pl.BlockSpec((DE, H), lambda i: (0, 0)),
            pl.BlockSpec((H, DE), lambda i: (0, 0)),
            pl.BlockSpec((1, DE), lambda i: (0, 0)),
        ],
        out_specs=[
            pl.BlockSpec((blk, DE), lambda i: (i, 0)),
            pl.BlockSpec((blk, AW), lambda i: (i, 0)),
        ],
        out_shape=[
            jax.ShapeDtypeStruct((E, DE), jnp.float32),
            jax.ShapeDtypeStruct((E, AW), jnp.float32),
        ],
    )(g, edge_attr, w_a, w2, b2.reshape(1, DE))


ST = 20
NPASS = 2
NPN = N // NPASS
APP = NPN * ST
SCH = 40


def _scatter_body(aug_hbm, sidx_hbm, out_hbm, aug_v, idx_v, acc, sem):
    wid = lax.axis_index("s") * NC + lax.axis_index("c")
    base = wid * EPW

    lanes = lax.iota(jnp.int32, 16)
    for p in range(NPASS):
        lo = p * APP

        def zero(i, _):
            acc[pl.ds(i * 16, 16)] = jnp.zeros((16,), jnp.float32)
            return 0

        lax.fori_loop(0, (APP + 16) // 16, zero, 0)

        def chunk(i, _):
            off = base + i * SCH
            pltpu.sync_copy(aug_hbm.at[pl.ds(off, SCH)], aug_v)
            pltpu.sync_copy(sidx_hbm.at[pl.ds(off, SCH)], idx_v)

            def edge(e4, _):
                for j in range(4):
                    e = e4 * 4 + j
                    for half in (0, 16):
                        rel = idx_v[e, pl.ds(half, 16)] - lo
                        val = aug_v[e, pl.ds(half, 16)]
                        m = plsc.bitcast(rel, jnp.uint32) < APP
                        idx = jnp.where(m, rel, APP + lanes)
                        plsc.addupdate_scatter(acc, [idx], val)
                return 0

            lax.fori_loop(0, SCH // 4, edge, 0)
            return 0

        lax.fori_loop(0, EPW // SCH, chunk, 0)
        pltpu.sync_copy(acc.at[pl.ds(0, APP)],
                        out_hbm.at[pl.ds(wid * (N * ST) + lo, APP)])


def _segment_sums(aug, sidx):
    mesh = plsc.VectorSubcoreMesh(core_axis_name="c", subcore_axis_name="s")
    return pl.kernel(
        _scatter_body,
        out_type=jax.ShapeDtypeStruct((NW * N * ST,), jnp.float32),
        mesh=mesh,
        compiler_params=pltpu.CompilerParams(needs_layout_passes=False),
        scratch_types=[
            pltpu.VMEM((SCH, AW), jnp.float32),
            pltpu.VMEM((SCH, AW), jnp.int32),
            pltpu.VMEM((APP + 16,), jnp.float32),
            pltpu.SemaphoreType.DMA,
        ],
    )(aug, sidx)


def _node_body(x_ref, p_ref, w1x_ref, w1a_ref, b1_ref, w2_ref, b2_ref, o_ref):
    parts = jnp.sum(p_ref[...], axis=0)
    sums = parts[:, :DE]
    cnts = parts[:, DE:DE + 1]
    agg = sums / jnp.maximum(cnts, 1.0)
    h = jax.nn.relu(
        jnp.dot(x_ref[...], w1x_ref[...], preferred_element_type=jnp.float32)
        + jnp.dot(agg, w1a_ref[...], preferred_element_type=jnp.float32)
        + b1_ref[...])
    o_ref[...] = jnp.dot(h, w2_ref[...], preferred_element_type=jnp.float32) + b2_ref[...]


def _node_mlp(x, parts, w1x, w1a, b1, w2, b2):
    blk = 1000
    return pl.pallas_call(
        _node_body,
        grid=(N // blk,),
        in_specs=[
            pl.BlockSpec((blk, D), lambda i: (i, 0)),
            pl.BlockSpec((NW, blk, ST), lambda i: (0, i, 0)),
            pl.BlockSpec((D, H), lambda i: (0, 0)),
            pl.BlockSpec((DE, H), lambda i: (0, 0)),
            pl.BlockSpec((1, H), lambda i: (0, 0)),
            pl.BlockSpec((H, D), lambda i: (0, 0)),
            pl.BlockSpec((1, D), lambda i: (0, 0)),
        ],
        out_specs=pl.BlockSpec((blk, D), lambda i: (i, 0)),
        out_shape=jax.ShapeDtypeStruct((N, D), jnp.float32),
    )(x, parts, w1x, w1a, b1.reshape(1, H), w2, b2.reshape(1, D))


@jax.jit
def kernel(x, edge_index, edge_attr, W1_e, b1_e, W2_e, b2_e,
           W1_n, b1_n, W2_n, b2_n):
    x = x.astype(jnp.float32)
    edge_attr = edge_attr.astype(jnp.float32)
    row = edge_index[0]
    col = edge_index[1]
    w_r = W1_e[:D]
    w_c = W1_e[D:2 * D]
    w_a = W1_e[2 * D:]

    cols = jnp.concatenate([jnp.arange(DE + 1, dtype=jnp.int32),
                            jnp.full((AW - DE - 1,), 1 << 30, jnp.int32)])
    sidx = row[:, None].astype(jnp.int32) * ST + cols[None, :]

    p_r, p_c = _precompute_tables(x, w_r, w_c, b1_e)
    g = _gather_add(p_r, p_c, row, col)
    emb, aug = _edge_mlp(g, edge_attr, w_a, W2_e, b2_e)
    parts = _segment_sums(aug, sidx).reshape(NW, N, ST)
    node = _node_mlp(x, parts, W1_n[:D], W1_n[D:], b1_n, W2_n, b2_n)
    return emb, node


# speedup vs baseline: 2.3752x; 1.9661x over previous
import jax
import jax.numpy as jnp
from jax import lax
from jax.experimental import pallas as pl
from jax.experimental.pallas import tpu as pltpu, tpu_sc as plsc

N = 10000
E = 160000
D = 256
DE = 16
H = 512

NC, NS = 2, 16
NW = NC * NS
EPW = E // NW
GCH = 96


def _pack_pairs(v):
    bits = lax.bitcast_convert_type(
        v.astype(jnp.bfloat16).astype(jnp.float32), jnp.int32)
    lo = lax.shift_right_logical(bits[:, :H // 2], 16)
    hi = jnp.bitwise_and(bits[:, H // 2:], jnp.int32(-65536))
    return lax.bitcast_convert_type(jnp.bitwise_or(hi, lo), jnp.float32)


def _mm1_body(x_ref, wr_ref, wc_ref, b_ref, pr_ref, pc_ref):
    xb = x_ref[...]
    pr = jnp.dot(xb, wr_ref[...], preferred_element_type=jnp.float32) + b_ref[...]
    pc = jnp.dot(xb, wc_ref[...], preferred_element_type=jnp.float32)
    pr_ref[...] = _pack_pairs(pr)
    pc_ref[...] = _pack_pairs(pc)


def _precompute_tables(x, w_r, w_c, b1):
    blk = 2000
    grid = (N // blk,)
    return pl.pallas_call(
        _mm1_body,
        grid=grid,
        in_specs=[
            pl.BlockSpec((blk, D), lambda i: (i, 0)),
            pl.BlockSpec((D, H), lambda i: (0, 0)),
            pl.BlockSpec((D, H), lambda i: (0, 0)),
            pl.BlockSpec((1, H), lambda i: (0, 0)),
        ],
        out_specs=[
            pl.BlockSpec((blk, H // 2), lambda i: (i, 0)),
            pl.BlockSpec((blk, H // 2), lambda i: (i, 0)),
        ],
        out_shape=[
            jax.ShapeDtypeStruct((N, H // 2), jnp.float32),
            jax.ShapeDtypeStruct((N, H // 2), jnp.float32),
        ],
    )(x, w_r, w_c, b1.reshape(1, H))


GTAIL = EPW - (EPW // GCH) * GCH


def _gather_body(pr_hbm, pc_hbm, row_hbm, col_hbm, out_hbm,
                 idx_r, idx_c, gr, gc, sem):
    wid = lax.axis_index("s") * NC + lax.axis_index("c")
    base = wid * EPW
    pltpu.sync_copy(row_hbm.at[pl.ds(base, EPW)], idx_r)
    pltpu.sync_copy(col_hbm.at[pl.ds(base, EPW)], idx_c)

    def do_chunk(o, rows):
        dr = gr.at[pl.ds(0, rows)]
        dc = gc.at[pl.ds(0, rows)]
        pltpu.async_copy(pr_hbm.at[idx_r.at[pl.ds(o, rows)]], dr, sem)
        pltpu.async_copy(pc_hbm.at[idx_c.at[pl.ds(o, rows)]], dc, sem).wait()
        pltpu.make_async_copy(pr_hbm.at[idx_r.at[pl.ds(o, rows)]], dr,
                              sem).wait()

        def add_row(e, _):
            for kk in range(0, H // 2, 16):
                sl = pl.ds(kk, 16)
                ss = (plsc.bitcast(gr[e, sl], jnp.bfloat16)
                      + plsc.bitcast(gc[e, sl], jnp.bfloat16))
                gr[e, sl] = plsc.bitcast(ss, jnp.float32)
            return 0

        lax.fori_loop(0, rows, add_row, 0)
        pltpu.sync_copy(dr, out_hbm.at[pl.ds(base + o, rows)])

    def chunk(i, _):
        do_chunk(i * GCH, GCH)
        return 0

    lax.fori_loop(0, EPW // GCH, chunk, 0)
    if GTAIL:
        do_chunk((EPW // GCH) * GCH, GTAIL)


def _gather_add(p_r, p_c, row, col):
    mesh = plsc.VectorSubcoreMesh(core_axis_name="c", subcore_axis_name="s")
    return pl.kernel(
        _gather_body,
        out_type=jax.ShapeDtypeStruct((E, H // 2), jnp.float32),
        mesh=mesh,
        compiler_params=pltpu.CompilerParams(needs_layout_passes=False),
        scratch_types=[
            pltpu.VMEM((EPW,), jnp.int32),
            pltpu.VMEM((EPW,), jnp.int32),
            pltpu.VMEM((GCH, H // 2), jnp.float32),
            pltpu.VMEM((GCH, H // 2), jnp.float32),
            pltpu.SemaphoreType.DMA,
        ],
    )(p_r, p_c, row, col)


def _edge_body(g_ref, ea_ref, wa_ref, w2_ref, b2_ref, emb_ref):
    a = jnp.dot(ea_ref[...].astype(jnp.bfloat16), wa_ref[...].astype(jnp.bfloat16),
                preferred_element_type=jnp.float32)
    u = lax.bitcast_convert_type(g_ref[...], jnp.int32)
    g_lo = lax.bitcast_convert_type(jnp.left_shift(u, 16), jnp.float32)
    g_hi = lax.bitcast_convert_type(
        jnp.bitwise_and(u, jnp.int32(-65536)), jnp.float32)
    g = jnp.concatenate([g_lo, g_hi], axis=1)
    h = jax.nn.relu(g + a)
    emb_ref[...] = jnp.dot(h.astype(jnp.bfloat16), w2_ref[...].astype(jnp.bfloat16),
                           preferred_element_type=jnp.float32) + b2_ref[...]


def _edge_mlp(g, edge_attr, w_a, w2, b2):
    blk = 2000
    return pl.pallas_call(
        _edge_body,
        grid=(E // blk,),
        in_specs=[
            pl.BlockSpec((blk, H // 2), lambda i: (i, 0)),
            pl.BlockSpec((blk, DE), lambda i: (i, 0)),
            pl.BlockSpec((DE, H), lambda i: (0, 0)),
            pl.BlockSpec((H, DE), lambda i: (0, 0)),
            pl.BlockSpec((1, DE), lambda i: (0, 0)),
        ],
        out_specs=pl.BlockSpec((blk, DE), lambda i: (i, 0)),
        out_shape=jax.ShapeDtypeStruct((E, DE), jnp.float32),
    )(g, edge_attr, w_a, w2, b2.reshape(1, DE))


NPASS = 2
NPN = N // NPASS
APP = NPN * DE
SCH = 200
EPWP = EPW + 8


def _scatter_body(emb_hbm, sidx_hbm, cidx_hbm, outs_hbm, outc_hbm,
                  emb_v, idx_v, cidx_v, acc, acc_c, sem):
    wid = lax.axis_index("s") * NC + lax.axis_index("c")
    base = wid * EPW

    pltpu.sync_copy(cidx_hbm.at[pl.ds(wid * EPWP, EPWP)], cidx_v)
    ones16 = jnp.full((16,), 1.0, jnp.float32)

    for p in range(NPASS):
        def zero(i, _):
            acc[pl.ds(i * 16, 16)] = jnp.zeros((16,), jnp.float32)
            return 0

        lax.fori_loop(0, (APP + 16) // 16, zero, 0)

        def zero_c(i, _):
            acc_c[pl.ds(i * 16, 16)] = jnp.zeros((16,), jnp.float32)
            return 0

        lax.fori_loop(0, (NPN + 16) // 16, zero_c, 0)

        def chunk(i, _):
            off = (base + i * SCH) * DE
            pltpu.sync_copy(emb_hbm.at[pl.ds(off, SCH * DE)], emb_v)
            pltpu.sync_copy(sidx_hbm.at[pl.ds(p * E * DE + off, SCH * DE)],
                            idx_v)

            def edge(e4, _):
                for j in range(4):
                    o = pl.ds((e4 * 4 + j) * DE, 16)
                    plsc.addupdate_scatter(acc, [idx_v[o]], emb_v[o])
                return 0

            lax.fori_loop(0, SCH // 4, edge, 0)
            return 0

        lax.fori_loop(0, EPW // SCH, chunk, 0)

        def cgrp(i, _):
            cidx = cidx_v[pl.ds(i * 16, 16)] - p * NPN
            m = plsc.bitcast(cidx, jnp.uint32) < NPN
            cidx = jnp.where(m, cidx, NPN + lax.iota(jnp.int32, 16))
            plsc.addupdate_scatter(acc_c, [cidx], ones16)
            return 0

        lax.fori_loop(0, EPWP // 16, cgrp, 0)

        lo = p * APP
        pltpu.sync_copy(acc.at[pl.ds(0, APP)],
                        outs_hbm.at[pl.ds(wid * (N * DE) + lo, APP)])
        pltpu.sync_copy(acc_c.at[pl.ds(0, NPN)],
                        outc_hbm.at[pl.ds(wid * N + p * NPN, NPN)])


def _segment_sums(emb, sidx, cidx):
    mesh = plsc.VectorSubcoreMesh(core_axis_name="c", subcore_axis_name="s")
    return pl.kernel(
        _scatter_body,
        out_type=[jax.ShapeDtypeStruct((NW * N * DE,), jnp.float32),
                  jax.ShapeDtypeStruct((NW * N,), jnp.float32)],
        mesh=mesh,
        compiler_params=pltpu.CompilerParams(needs_layout_passes=False),
        scratch_types=[
            pltpu.VMEM((SCH * DE,), jnp.float32),
            pltpu.VMEM((SCH * DE,), jnp.int32),
            pltpu.VMEM((EPWP,), jnp.int32),
            pltpu.VMEM((APP + 16,), jnp.float32),
            pltpu.VMEM((NPN + 16,), jnp.float32),
            pltpu.SemaphoreType.DMA,
        ],
    )(emb, sidx, cidx)


def _rsum_body(p_ref, o_ref):
    o_ref[...] = jnp.sum(p_ref[...], axis=0)


def _reduce_sums(parts):
    return pl.pallas_call(
        _rsum_body,
        in_specs=[pl.BlockSpec((NW, N * DE), lambda: (0, 0))],
        out_specs=pl.BlockSpec((N * DE,), lambda: (0,)),
        out_shape=jax.ShapeDtypeStruct((N * DE,), jnp.float32),
    )(parts)


def _rcnt_body(p_ref, o_ref):
    o_ref[...] = jnp.sum(p_ref[...], axis=0)[:, None]


def _reduce_counts(parts):
    return pl.pallas_call(
        _rcnt_body,
        in_specs=[pl.BlockSpec((NW, N), lambda: (0, 0))],
        out_specs=pl.BlockSpec((N, 1), lambda: (0, 0)),
        out_shape=jax.ShapeDtypeStruct((N, 1), jnp.float32),
    )(parts)


def _node_body(x_ref, ps_ref, pc_ref, w1x_ref, w1a_ref, b1_ref, w2_ref,
               b2_ref, o_ref):
    sums = ps_ref[...]
    cnts = pc_ref[...]
    agg = sums / jnp.maximum(cnts, 1.0)
    h = jax.nn.relu(
        jnp.dot(x_ref[...], w1x_ref[...], preferred_element_type=jnp.float32)
        + jnp.dot(agg, w1a_ref[...], preferred_element_type=jnp.float32)
        + b1_ref[...])
    o_ref[...] = jnp.dot(h, w2_ref[...], preferred_element_type=jnp.float32) + b2_ref[...]


def _node_mlp(x, parts_s, parts_c, w1x, w1a, b1, w2, b2):
    blk = 1000
    return pl.pallas_call(
        _node_body,
        grid=(N // blk,),
        in_specs=[
            pl.BlockSpec((blk, D), lambda i: (i, 0)),
            pl.BlockSpec((blk, DE), lambda i: (i, 0)),
            pl.BlockSpec((blk, 1), lambda i: (i, 0)),
            pl.BlockSpec((D, H), lambda i: (0, 0)),
            pl.BlockSpec((DE, H), lambda i: (0, 0)),
            pl.BlockSpec((1, H), lambda i: (0, 0)),
            pl.BlockSpec((H, D), lambda i: (0, 0)),
            pl.BlockSpec((1, D), lambda i: (0, 0)),
        ],
        out_specs=pl.BlockSpec((blk, D), lambda i: (i, 0)),
        out_shape=jax.ShapeDtypeStruct((N, D), jnp.float32),
    )(x, parts_s, parts_c, w1x, w1a, b1.reshape(1, H), w2, b2.reshape(1, D))


@jax.jit
def kernel(x, edge_index, edge_attr, W1_e, b1_e, W2_e, b2_e,
           W1_n, b1_n, W2_n, b2_n):
    x = x.astype(jnp.float32)
    edge_attr = edge_attr.astype(jnp.float32)
    row = edge_index[0]
    col = edge_index[1]
    w_r = W1_e[:D]
    w_c = W1_e[D:2 * D]
    w_a = W1_e[2 * D:]

    iota16 = jnp.arange(DE, dtype=jnp.int32)
    sidx = jnp.stack([
        jnp.where((row >= p * NPN) & (row < (p + 1) * NPN),
                  (row - p * NPN) * DE, APP)[:, None] + iota16[None, :]
        for p in range(NPASS)
    ]).reshape(NPASS * E * DE)
    cidx = jnp.pad(row.reshape(NW, EPW), ((0, 0), (0, EPWP - EPW)),
                   constant_values=N).reshape(NW * EPWP)

    perm = jnp.concatenate([jnp.arange(0, H, 2), jnp.arange(1, H, 2)])
    p_r, p_c = _precompute_tables(x, w_r[:, perm], w_c[:, perm], b1_e[perm])
    g = _gather_add(p_r, p_c, row, col)
    emb = _edge_mlp(g, edge_attr, w_a[:, perm], W2_e[perm, :], b2_e)
    parts_s, parts_c = _segment_sums(emb.reshape(E * DE), sidx, cidx)
    sums = _reduce_sums(parts_s.reshape(NW, N * DE)).reshape(N, DE)
    cnts = _reduce_counts(parts_c.reshape(NW, N))
    node = _node_mlp(x, sums, cnts,
                     W1_n[:D], W1_n[D:], b1_n, W2_n, b2_n)
    return emb, node
